# baseline (device time: 65532 ns/iter reference)
import os

import jax
import jax.numpy as jnp
from jax import lax
from jax.experimental import pallas as pl
from jax.experimental.pallas import tpu as pltpu

KMODE = os.environ.get("KMODE", "full")

N_DEV = 16
B, SQ, DM = 2, 512, 768
DH = 64
BLK = 64
ROWS = B * SQ
CH = ROWS // N_DEV


def kernel(x, Wq, K_ext, V_ext, Wo):
    H = K_ext.shape[2]
    HD = H * DH

    x2 = x.reshape(ROWS, DM)

    def body(x_ref, wq_any, k_ref, v_ref, wo_any, o_ref,
             wqf_ref, wof_ref, wq_ref, wo_ref,
             xb_ref, kb_ref, vb_ref, q_ref, ctx_ref, part_ref, red_ref,
             rs_recv, agbuf,
             copy_sems, rs_ssem, rs_rsem, ag_ssem, ag_rsem):
        me = lax.axis_index("i")

        wq_dma = pltpu.make_async_copy(
            wq_any.at[:, pl.ds(me * HD, HD)], wqf_ref, copy_sems.at[0])
        wo_dma = pltpu.make_async_copy(
            wo_any.at[pl.ds(me * HD, HD), :], wof_ref, copy_sems.at[1])
        wq_dma.start()
        wo_dma.start()

        xb_ref[...] = x_ref[...].astype(jnp.bfloat16)
        for b in range(B):
            for h in range(H):
                cols = slice(h * DH, (h + 1) * DH)
                kb_ref[b * SQ:(b + 1) * SQ, cols] = (
                    k_ref[b, :, h, :].astype(jnp.bfloat16))
                vb_ref[b * SQ:(b + 1) * SQ, cols] = (
                    v_ref[b, :, h, :].astype(jnp.bfloat16))

        wq_dma.wait()
        wo_dma.wait()
        wq_ref[...] = wqf_ref[...].astype(jnp.bfloat16)
        wo_ref[...] = wof_ref[...].astype(jnp.bfloat16)

        q_ref[...] = jnp.dot(
            xb_ref[...], wq_ref[...], preferred_element_type=jnp.float32
        ).astype(jnp.bfloat16)

        qb = lax.broadcasted_iota(jnp.int32, (SQ, SQ), 0) // BLK
        kb = lax.broadcasted_iota(jnp.int32, (SQ, SQ), 1) // BLK
        mask = (qb == kb) | (kb == 0) | ((qb + kb) % 3 == 0)
        bias = jnp.where(mask, 0.0, -1e9).astype(jnp.float32)

        for b in range(B):
            for h in range(H):
                if KMODE == "noattn":
                    break
                cols = slice(h * DH, (h + 1) * DH)
                rows = slice(b * SQ, (b + 1) * SQ)
                s = lax.dot_general(
                    q_ref[rows, cols], kb_ref[rows, cols],
                    (((1,), (1,)), ((), ())),
                    preferred_element_type=jnp.float32,
                )
                e = jnp.exp(s * 0.125 + bias)
                rsum = jnp.sum(e, axis=1, keepdims=True)
                ctx = jnp.dot(e.astype(jnp.bfloat16), vb_ref[rows, cols],
                              preferred_element_type=jnp.float32)
                ctx = ctx * (1.0 / rsum)
                ctx_ref[rows, cols] = ctx.astype(jnp.bfloat16)

        if KMODE == "nocomm":
            for o in range(N_DEV):
                j = (me + o) % N_DEV
                rows = pl.ds(j * CH, CH)
                pc = jnp.dot(ctx_ref[rows, :], wo_ref[...],
                             preferred_element_type=jnp.float32)
                o_ref[rows, :] = pc
            return

        bar = pltpu.get_barrier_semaphore()
        for j in range(N_DEV):
            pl.semaphore_signal(bar, inc=1, device_id=(j,),
                                device_id_type=pl.DeviceIdType.MESH)
        pl.semaphore_wait(bar, N_DEV)

        rs_descs = []
        for o in range(N_DEV):
            j = (me + o) % N_DEV
            rows = pl.ds(j * CH, CH)
            pc = jnp.dot(ctx_ref[rows, :], wo_ref[...],
                         preferred_element_type=jnp.float32).astype(jnp.bfloat16)
            part_ref[rows, :] = pc
            if o == 0:
                rs_recv[me, :, :] = pc
            else:
                d = pltpu.make_async_remote_copy(
                    src_ref=part_ref.at[rows, :],
                    dst_ref=rs_recv.at[me],
                    send_sem=rs_ssem.at[o - 1],
                    recv_sem=rs_rsem.at[o - 1],
                    device_id=(j,), device_id_type=pl.DeviceIdType.MESH)
                d.start()
                rs_descs.append(d)

        acc = rs_recv[me].astype(jnp.float32)
        for o in range(1, N_DEV):
            rs_descs[o - 1].wait_recv()
            acc = acc + rs_recv[(me - o) % N_DEV].astype(jnp.float32)
        red_ref[...] = acc.astype(jnp.bfloat16)

        myrows = pl.ds(me * CH, CH)
        agbuf[myrows, :] = red_ref[...]
        o_ref[myrows, :] = acc
        ag_descs = []
        for o in range(1, N_DEV):
            j = (me + o) % N_DEV
            d = pltpu.make_async_remote_copy(
                src_ref=red_ref,
                dst_ref=agbuf.at[myrows, :],
                send_sem=ag_ssem.at[o - 1],
                recv_sem=ag_rsem.at[o - 1],
                device_id=(j,), device_id_type=pl.DeviceIdType.MESH)
            d.start()
            ag_descs.append(d)
        for d in rs_descs:
            d.wait_send()
        for o in range(1, N_DEV):
            ag_descs[o - 1].wait_recv()
            rows = pl.ds(((me - o) % N_DEV) * CH, CH)
            o_ref[rows, :] = agbuf[rows, :].astype(jnp.float32)
        for d in ag_descs:
            d.wait_send()

    out = pl.pallas_call(
        body,
        out_shape=jax.ShapeDtypeStruct((ROWS, DM), jnp.float32),
        in_specs=[
            pl.BlockSpec(memory_space=pltpu.VMEM),
            pl.BlockSpec(memory_space=pl.ANY),
            pl.BlockSpec(memory_space=pltpu.VMEM),
            pl.BlockSpec(memory_space=pltpu.VMEM),
            pl.BlockSpec(memory_space=pl.ANY),
        ],
        out_specs=pl.BlockSpec(memory_space=pltpu.VMEM),
        scratch_shapes=[
            pltpu.VMEM((DM, HD), jnp.float32),
            pltpu.VMEM((HD, DM), jnp.float32),
            pltpu.VMEM((DM, HD), jnp.bfloat16),
            pltpu.VMEM((HD, DM), jnp.bfloat16),
            pltpu.VMEM((ROWS, DM), jnp.bfloat16),
            pltpu.VMEM((ROWS, HD), jnp.bfloat16),
            pltpu.VMEM((ROWS, HD), jnp.bfloat16),
            pltpu.VMEM((ROWS, HD), jnp.bfloat16),
            pltpu.VMEM((ROWS, HD), jnp.bfloat16),
            pltpu.VMEM((ROWS, DM), jnp.bfloat16),
            pltpu.VMEM((CH, DM), jnp.bfloat16),
            pltpu.VMEM((N_DEV, CH, DM), jnp.bfloat16),
            pltpu.VMEM((ROWS, DM), jnp.bfloat16),
            pltpu.SemaphoreType.DMA((2,)),
            pltpu.SemaphoreType.DMA((N_DEV - 1,)),
            pltpu.SemaphoreType.DMA((N_DEV - 1,)),
            pltpu.SemaphoreType.DMA((N_DEV - 1,)),
            pltpu.SemaphoreType.DMA((N_DEV - 1,)),
        ],
        compiler_params=pltpu.CompilerParams(
            collective_id=None if KMODE == "nocomm" else 0),
    )(x2, Wq, K_ext, V_ext, Wo)

    return out.reshape(B, SQ, DM)


# device time: 61842 ns/iter; 1.0597x vs baseline; 1.0597x over previous
import os

import jax
import jax.numpy as jnp
from jax import lax
from jax.experimental import pallas as pl
from jax.experimental.pallas import tpu as pltpu

KMODE = os.environ.get("KMODE", "full")

N_DEV = 16
B, SQ, DM = 2, 512, 768
DH = 64
BLK = 64
ROWS = B * SQ
CH = ROWS // N_DEV


def kernel(x, Wq, K_ext, V_ext, Wo):
    H = K_ext.shape[2]
    HD = H * DH

    x2 = x.reshape(ROWS, DM)

    def body(x_ref, wq_any, k_any, v_any, wo_any, o_ref,
             wqf_ref, wof_ref, wq_ref, wo_ref,
             xb_ref, kf_ref, vf_ref, q_ref, ctx_ref, part_ref, red_ref,
             rs_recv, agbuf,
             copy_sems, kv_sems, rs_ssem, rs_rsem, ag_ssem, ag_rsem):
        me = lax.axis_index("i")

        wq_dma = pltpu.make_async_copy(
            wq_any.at[:, pl.ds(me * HD, HD)], wqf_ref, copy_sems.at[0])
        wo_dma = pltpu.make_async_copy(
            wo_any.at[pl.ds(me * HD, HD), :], wof_ref, copy_sems.at[1])
        wq_dma.start()
        wo_dma.start()
        kv_dmas = []
        for b in range(B):
            for h in range(H):
                i = b * H + h
                dk = pltpu.make_async_copy(
                    k_any.at[b, :, h, :], kf_ref.at[i], kv_sems.at[2 * i])
                dv = pltpu.make_async_copy(
                    v_any.at[b, :, h, :], vf_ref.at[i], kv_sems.at[2 * i + 1])
                dk.start()
                dv.start()
                kv_dmas += [dk, dv]

        xb_ref[...] = x_ref[...].astype(jnp.bfloat16)

        wq_dma.wait()
        wo_dma.wait()
        wq_ref[...] = wqf_ref[...].astype(jnp.bfloat16)
        wo_ref[...] = wof_ref[...].astype(jnp.bfloat16)

        q_ref[...] = jnp.dot(
            xb_ref[...], wq_ref[...], preferred_element_type=jnp.float32
        ).astype(jnp.bfloat16)

        qb = lax.broadcasted_iota(jnp.int32, (SQ, SQ), 0) // BLK
        kb = lax.broadcasted_iota(jnp.int32, (SQ, SQ), 1) // BLK
        mask = (qb == kb) | (kb == 0) | ((qb + kb) % 3 == 0)
        bias = jnp.where(mask, 0.0, -1e9).astype(jnp.float32)

        for d in kv_dmas:
            d.wait()
        for b in range(B):
            for h in range(H):
                if KMODE == "noattn":
                    break
                i = b * H + h
                cols = slice(h * DH, (h + 1) * DH)
                rows = slice(b * SQ, (b + 1) * SQ)
                s = lax.dot_general(
                    q_ref[rows, cols], kf_ref[i].astype(jnp.bfloat16),
                    (((1,), (1,)), ((), ())),
                    preferred_element_type=jnp.float32,
                )
                e = jnp.exp(s * 0.125 + bias)
                rsum = jnp.sum(e, axis=1, keepdims=True)
                ctx = jnp.dot(e.astype(jnp.bfloat16),
                              vf_ref[i].astype(jnp.bfloat16),
                              preferred_element_type=jnp.float32)
                ctx = ctx * (1.0 / rsum)
                ctx_ref[rows, cols] = ctx.astype(jnp.bfloat16)

        if KMODE == "nocomm":
            for o in range(N_DEV):
                j = (me + o) % N_DEV
                rows = pl.ds(j * CH, CH)
                pc = jnp.dot(ctx_ref[rows, :], wo_ref[...],
                             preferred_element_type=jnp.float32)
                o_ref[rows, :] = pc
            return

        bar = pltpu.get_barrier_semaphore()
        for j in range(N_DEV):
            pl.semaphore_signal(bar, inc=1, device_id=(j,),
                                device_id_type=pl.DeviceIdType.MESH)
        pl.semaphore_wait(bar, N_DEV)

        rs_descs = []
        for o in range(N_DEV):
            j = (me + o) % N_DEV
            rows = pl.ds(j * CH, CH)
            pc = jnp.dot(ctx_ref[rows, :], wo_ref[...],
                         preferred_element_type=jnp.float32).astype(jnp.bfloat16)
            part_ref[rows, :] = pc
            if o == 0:
                rs_recv[me, :, :] = pc
            else:
                d = pltpu.make_async_remote_copy(
                    src_ref=part_ref.at[rows, :],
                    dst_ref=rs_recv.at[me],
                    send_sem=rs_ssem.at[o - 1],
                    recv_sem=rs_rsem.at[o - 1],
                    device_id=(j,), device_id_type=pl.DeviceIdType.MESH)
                d.start()
                rs_descs.append(d)

        acc = rs_recv[me].astype(jnp.float32)
        for o in range(1, N_DEV):
            rs_descs[o - 1].wait_recv()
            acc = acc + rs_recv[(me - o) % N_DEV].astype(jnp.float32)
        red_ref[...] = acc.astype(jnp.bfloat16)

        myrows = pl.ds(me * CH, CH)
        agbuf[myrows, :] = red_ref[...]
        o_ref[myrows, :] = acc
        ag_descs = []
        for o in range(1, N_DEV):
            j = (me + o) % N_DEV
            d = pltpu.make_async_remote_copy(
                src_ref=red_ref,
                dst_ref=agbuf.at[myrows, :],
                send_sem=ag_ssem.at[o - 1],
                recv_sem=ag_rsem.at[o - 1],
                device_id=(j,), device_id_type=pl.DeviceIdType.MESH)
            d.start()
            ag_descs.append(d)
        for d in rs_descs:
            d.wait_send()
        for o in range(1, N_DEV):
            ag_descs[o - 1].wait_recv()
            rows = pl.ds(((me - o) % N_DEV) * CH, CH)
            o_ref[rows, :] = agbuf[rows, :].astype(jnp.float32)
        for d in ag_descs:
            d.wait_send()

    out = pl.pallas_call(
        body,
        out_shape=jax.ShapeDtypeStruct((ROWS, DM), jnp.float32),
        in_specs=[
            pl.BlockSpec(memory_space=pltpu.VMEM),
            pl.BlockSpec(memory_space=pl.ANY),
            pl.BlockSpec(memory_space=pl.ANY),
            pl.BlockSpec(memory_space=pl.ANY),
            pl.BlockSpec(memory_space=pl.ANY),
        ],
        out_specs=pl.BlockSpec(memory_space=pltpu.VMEM),
        scratch_shapes=[
            pltpu.VMEM((DM, HD), jnp.float32),
            pltpu.VMEM((HD, DM), jnp.float32),
            pltpu.VMEM((DM, HD), jnp.bfloat16),
            pltpu.VMEM((HD, DM), jnp.bfloat16),
            pltpu.VMEM((ROWS, DM), jnp.bfloat16),
            pltpu.VMEM((B * H, SQ, DH), jnp.float32),
            pltpu.VMEM((B * H, SQ, DH), jnp.float32),
            pltpu.VMEM((ROWS, HD), jnp.bfloat16),
            pltpu.VMEM((ROWS, HD), jnp.bfloat16),
            pltpu.VMEM((ROWS, DM), jnp.bfloat16),
            pltpu.VMEM((CH, DM), jnp.bfloat16),
            pltpu.VMEM((N_DEV, CH, DM), jnp.bfloat16),
            pltpu.VMEM((ROWS, DM), jnp.bfloat16),
            pltpu.SemaphoreType.DMA((2,)),
            pltpu.SemaphoreType.DMA((2 * B * H,)),
            pltpu.SemaphoreType.DMA((N_DEV - 1,)),
            pltpu.SemaphoreType.DMA((N_DEV - 1,)),
            pltpu.SemaphoreType.DMA((N_DEV - 1,)),
            pltpu.SemaphoreType.DMA((N_DEV - 1,)),
        ],
        compiler_params=pltpu.CompilerParams(
            collective_id=None if KMODE == "nocomm" else 0),
    )(x2, Wq, K_ext, V_ext, Wo)

    return out.reshape(B, SQ, DM)


# device time: 59037 ns/iter; 1.1100x vs baseline; 1.0475x over previous
import os

import jax
import jax.numpy as jnp
from jax import lax
from jax.experimental import pallas as pl
from jax.experimental.pallas import tpu as pltpu

KMODE = os.environ.get("KMODE", "full")

N_DEV = 16
B, SQ, DM = 2, 512, 768
DH = 64
BLK = 64
ROWS = B * SQ
SC = SQ // N_DEV


def kernel(x, Wq, K_ext, V_ext, Wo):
    H = K_ext.shape[2]
    HD = H * DH

    idx = lax.axis_index("i")
    x2 = x.reshape(ROWS, DM)
    k2 = K_ext.reshape(ROWS, HD)
    v2 = V_ext.reshape(ROWS, HD)
    wq_s = lax.dynamic_slice(Wq, (0, idx * HD), (DM, HD)).astype(jnp.bfloat16)
    wo_s = lax.dynamic_slice(Wo, (idx * HD, 0), (HD, DM)).astype(jnp.bfloat16)

    def body(x_ref, wq_ref, k_ref, v_ref, wo_ref, o_ref,
             xb_ref, kb_ref, vb_ref, q_ref, ctx_ref, part_ref,
             redA_ref, redB_ref, recvA, recvB, agbuf,
             rsA_s, rsA_r, rsB_s, rsB_r, agA_s, agA_r, agB_s, agB_r):
        me = lax.axis_index("i")

        xb_ref[...] = x_ref[...].astype(jnp.bfloat16)
        kb_ref[...] = k_ref[...].astype(jnp.bfloat16)
        vb_ref[...] = v_ref[...].astype(jnp.bfloat16)

        q_ref[...] = jnp.dot(
            xb_ref[...], wq_ref[...], preferred_element_type=jnp.float32
        ).astype(jnp.bfloat16)

        qb = lax.broadcasted_iota(jnp.int32, (SQ, SQ), 0) // BLK
        kb = lax.broadcasted_iota(jnp.int32, (SQ, SQ), 1) // BLK
        mask = (qb == kb) | (kb == 0) | ((qb + kb) % 3 == 0)
        bias = jnp.where(mask, 0.0, -1e9).astype(jnp.float32)

        def attention(b):
            rows = slice(b * SQ, (b + 1) * SQ)
            for h in range(H):
                if KMODE == "noattn":
                    break
                cols = slice(h * DH, (h + 1) * DH)
                s = lax.dot_general(
                    q_ref[rows, cols], kb_ref[rows, cols],
                    (((1,), (1,)), ((), ())),
                    preferred_element_type=jnp.float32,
                )
                e = jnp.exp(s * 0.125 + bias)
                rsum = jnp.sum(e, axis=1, keepdims=True)
                ctx = jnp.dot(e.astype(jnp.bfloat16), vb_ref[rows, cols],
                              preferred_element_type=jnp.float32)
                ctx = ctx * (1.0 / rsum)
                ctx_ref[rows, cols] = ctx.astype(jnp.bfloat16)

        def outproj(base, recv, ssem, rsem):
            descs = []
            for o in range(N_DEV):
                j = (me + o) % N_DEV
                rows = pl.ds(base + j * SC, SC)
                pc = jnp.dot(ctx_ref[rows, :], wo_ref[...],
                             preferred_element_type=jnp.float32
                             ).astype(jnp.bfloat16)
                part_ref[rows, :] = pc
                if o == 0:
                    recv[me, :, :] = pc
                else:
                    d = pltpu.make_async_remote_copy(
                        src_ref=part_ref.at[rows, :],
                        dst_ref=recv.at[me],
                        send_sem=ssem.at[o - 1],
                        recv_sem=rsem.at[o - 1],
                        device_id=(j,), device_id_type=pl.DeviceIdType.MESH)
                    d.start()
                    descs.append(d)
            return descs

        def reduce_chunk(recv, descs, red_ref):
            acc = recv[me].astype(jnp.float32)
            for o in range(1, N_DEV):
                descs[o - 1].wait_recv()
                acc = acc + recv[(me - o) % N_DEV].astype(jnp.float32)
            red_ref[...] = acc.astype(jnp.bfloat16)
            return acc

        def ag_push(base, red_ref, acc, ssem, rsem):
            myrows = pl.ds(base + me * SC, SC)
            agbuf[myrows, :] = red_ref[...]
            o_ref[myrows, :] = acc
            descs = []
            for o in range(1, N_DEV):
                j = (me + o) % N_DEV
                d = pltpu.make_async_remote_copy(
                    src_ref=red_ref,
                    dst_ref=agbuf.at[myrows, :],
                    send_sem=ssem.at[o - 1],
                    recv_sem=rsem.at[o - 1],
                    device_id=(j,), device_id_type=pl.DeviceIdType.MESH)
                d.start()
                descs.append(d)
            return descs

        def ag_collect(base, descs):
            for o in range(1, N_DEV):
                descs[o - 1].wait_recv()
                rows = pl.ds(base + ((me - o) % N_DEV) * SC, SC)
                o_ref[rows, :] = agbuf[rows, :].astype(jnp.float32)

        attention(0)

        if KMODE == "nocomm":
            for b in range(1, B):
                attention(b)
            for o in range(N_DEV):
                j = (me + o) % N_DEV
                for base in (0, SQ):
                    rows = pl.ds(base + j * SC, SC)
                    o_ref[rows, :] = jnp.dot(
                        ctx_ref[rows, :], wo_ref[...],
                        preferred_element_type=jnp.float32)
            return

        bar = pltpu.get_barrier_semaphore()
        for j in range(N_DEV):
            pl.semaphore_signal(bar, inc=1, device_id=(j,),
                                device_id_type=pl.DeviceIdType.MESH)
        pl.semaphore_wait(bar, N_DEV)

        rsA = outproj(0, recvA, rsA_s, rsA_r)
        attention(1)
        rsB = outproj(SQ, recvB, rsB_s, rsB_r)
        accA = reduce_chunk(recvA, rsA, redA_ref)
        agA = ag_push(0, redA_ref, accA, agA_s, agA_r)
        accB = reduce_chunk(recvB, rsB, redB_ref)
        agB = ag_push(SQ, redB_ref, accB, agB_s, agB_r)

        for d in rsA + rsB:
            d.wait_send()
        ag_collect(0, agA)
        ag_collect(SQ, agB)
        for d in agA + agB:
            d.wait_send()

    out = pl.pallas_call(
        body,
        out_shape=jax.ShapeDtypeStruct((ROWS, DM), jnp.float32),
        in_specs=[pl.BlockSpec(memory_space=pltpu.VMEM)] * 5,
        out_specs=pl.BlockSpec(memory_space=pltpu.VMEM),
        scratch_shapes=[
            pltpu.VMEM((ROWS, DM), jnp.bfloat16),
            pltpu.VMEM((ROWS, HD), jnp.bfloat16),
            pltpu.VMEM((ROWS, HD), jnp.bfloat16),
            pltpu.VMEM((ROWS, HD), jnp.bfloat16),
            pltpu.VMEM((ROWS, HD), jnp.bfloat16),
            pltpu.VMEM((ROWS, DM), jnp.bfloat16),
            pltpu.VMEM((SC, DM), jnp.bfloat16),
            pltpu.VMEM((SC, DM), jnp.bfloat16),
            pltpu.VMEM((N_DEV, SC, DM), jnp.bfloat16),
            pltpu.VMEM((N_DEV, SC, DM), jnp.bfloat16),
            pltpu.VMEM((ROWS, DM), jnp.bfloat16),
            pltpu.SemaphoreType.DMA((N_DEV - 1,)),
            pltpu.SemaphoreType.DMA((N_DEV - 1,)),
            pltpu.SemaphoreType.DMA((N_DEV - 1,)),
            pltpu.SemaphoreType.DMA((N_DEV - 1,)),
            pltpu.SemaphoreType.DMA((N_DEV - 1,)),
            pltpu.SemaphoreType.DMA((N_DEV - 1,)),
            pltpu.SemaphoreType.DMA((N_DEV - 1,)),
            pltpu.SemaphoreType.DMA((N_DEV - 1,)),
        ],
        compiler_params=pltpu.CompilerParams(
            collective_id=None if KMODE == "nocomm" else 0),
    )(x2, wq_s, k2, v2, wo_s)

    return out.reshape(B, SQ, DM)
